# baseline (speedup 1.0000x reference)
import functools

import jax
import jax.numpy as jnp
from jax import lax
from jax.experimental import pallas as pl
from jax.experimental.pallas import tpu as pltpu
from jax.experimental.pallas import tpu_sc as plsc

_B = 8192
_D = 2048
_NC = 2
_NS = 16
_NW = _NC * _NS
_ROWS_PER_W = _B // _NW
_CH = 16
_NCHUNK = _ROWS_PER_W // _CH
_NBUF = 3


def _pos_gather_body(lim_hbm, table_hbm, out_hbm, lim_v,
                     idx0, idx1, idx2, rows0, rows1, rows2,
                     gsem0, gsem1, gsem2, ssem0, ssem1, ssem2):
    wid = lax.axis_index("s") * _NC + lax.axis_index("c")
    base = wid * _ROWS_PER_W

    pltpu.sync_copy(lim_hbm, lim_v)
    lim_vec = lim_v[...]
    limit = lim_vec[0]
    lanes = lax.iota(jnp.int32, _CH)

    idx_refs = (idx0, idx1, idx2)
    row_refs = (rows0, rows1, rows2)
    gsems = (gsem0, gsem1, gsem2)
    ssems = (ssem0, ssem1, ssem2)

    all_identity = base + _ROWS_PER_W - 1 <= limit

    @pl.when(all_identity)
    def _fast():
        pltpu.async_copy(table_hbm.at[pl.ds(base, _ROWS_PER_W)],
                         out_hbm.at[pl.ds(base, _ROWS_PER_W)],
                         gsems[0]).wait()

    @pl.when(jnp.logical_not(all_identity))
    def _general():
        def fire_gather(g):
            slot = g % _NBUF
            row0 = base + g * _CH
            idx_refs[slot][...] = jnp.minimum(row0 + lanes, lim_vec)
            return pltpu.async_copy(table_hbm.at[idx_refs[slot]],
                                    row_refs[slot], gsems[slot])

        gathers = [None] * _NBUF
        stores = [None] * _NBUF
        for b in range(_NBUF):
            gathers[b] = fire_gather(b)
        for g in range(_NCHUNK):
            slot = g % _NBUF
            gathers[slot].wait()
            stores[slot] = pltpu.async_copy(
                row_refs[slot], out_hbm.at[pl.ds(base + g * _CH, _CH)],
                ssems[slot])
            nxt = g + _NBUF
            if nxt < _NCHUNK:
                stores[slot].wait()
                gathers[slot] = fire_gather(nxt)
            elif g >= _NCHUNK - _NBUF:
                stores[slot].wait()


_pos_gather = functools.partial(
    pl.kernel,
    out_type=jax.ShapeDtypeStruct((_B, _D), jnp.float32),
    mesh=plsc.VectorSubcoreMesh(core_axis_name="c", subcore_axis_name="s"),
    scratch_types=[
        pltpu.VMEM((_CH,), jnp.int32),
        pltpu.VMEM((_CH,), jnp.int32),
        pltpu.VMEM((_CH,), jnp.int32),
        pltpu.VMEM((_CH,), jnp.int32),
        pltpu.VMEM((_CH, _D), jnp.float32),
        pltpu.VMEM((_CH, _D), jnp.float32),
        pltpu.VMEM((_CH, _D), jnp.float32),
        pltpu.SemaphoreType.DMA,
        pltpu.SemaphoreType.DMA,
        pltpu.SemaphoreType.DMA,
        pltpu.SemaphoreType.DMA,
        pltpu.SemaphoreType.DMA,
        pltpu.SemaphoreType.DMA,
    ],
)(_pos_gather_body)


def kernel(seq_len, table):
    lim = jnp.full((_CH,), jnp.asarray(seq_len, jnp.int32) - 1, jnp.int32)
    out = _pos_gather(lim, table)
    return out[None]


# speedup vs baseline: 21.5114x; 21.5114x over previous
import functools

import jax
import jax.numpy as jnp
from jax import lax
from jax.experimental import pallas as pl
from jax.experimental.pallas import tpu as pltpu

_B = 8192
_D = 2048
_R = 512
_NBLK = _B // _R


def _tc_body(lim_ref, table_ref, limrow_ref, out_ref):
    j = pl.program_id(0)
    limit = lim_ref[0]
    row0 = j * _R
    full_identity = row0 + _R - 1 <= limit

    @pl.when(full_identity)
    def _():
        out_ref[...] = table_ref[...]

    @pl.when(jnp.logical_not(full_identity))
    def _():
        rows = row0 + lax.broadcasted_iota(jnp.int32, (_R, 1), 0)
        limrow = limrow_ref[0]
        out_ref[...] = jnp.where(rows <= limit, table_ref[...], limrow)


_tc_copy = pl.pallas_call(
    _tc_body,
    grid_spec=pltpu.PrefetchScalarGridSpec(
        num_scalar_prefetch=1,
        grid=(_NBLK,),
        in_specs=[
            pl.BlockSpec((_R, _D), lambda j, lim: (j, 0)),
            pl.BlockSpec((1, 1, _D), lambda j, lim: (lim[0], 0, 0)),
        ],
        out_specs=pl.BlockSpec((_R, _D), lambda j, lim: (j, 0)),
    ),
    out_shape=jax.ShapeDtypeStruct((_B, _D), jnp.float32),
)


def kernel(seq_len, table):
    lim = jnp.asarray(seq_len, jnp.int32).reshape(1) - 1
    out = _tc_copy(lim, table, table[:, None, :])
    return out[None]


# speedup vs baseline: 21.8149x; 1.0141x over previous
import functools

import jax
import jax.numpy as jnp
from jax import lax
from jax.experimental import pallas as pl
from jax.experimental.pallas import tpu as pltpu

_B = 8192
_D = 2048
_R = 1024
_NBLK = _B // _R


def _tc_body(lim_ref, table_ref, limrow_ref, out_ref):
    j = pl.program_id(0)
    limit = lim_ref[0]
    row0 = j * _R
    full_identity = row0 + _R - 1 <= limit

    @pl.when(full_identity)
    def _():
        out_ref[...] = table_ref[...]

    @pl.when(jnp.logical_not(full_identity))
    def _():
        rows = row0 + lax.broadcasted_iota(jnp.int32, (_R, 1), 0)
        limrow = limrow_ref[0]
        out_ref[...] = jnp.where(rows <= limit, table_ref[...], limrow)


_tc_copy = pl.pallas_call(
    _tc_body,
    grid_spec=pltpu.PrefetchScalarGridSpec(
        num_scalar_prefetch=1,
        grid=(_NBLK,),
        in_specs=[
            pl.BlockSpec((_R, _D), lambda j, lim: (j, 0)),
            pl.BlockSpec((1, 1, _D), lambda j, lim: (lim[0], 0, 0)),
        ],
        out_specs=pl.BlockSpec((_R, _D), lambda j, lim: (j, 0)),
    ),
    out_shape=jax.ShapeDtypeStruct((_B, _D), jnp.float32),
)


def kernel(seq_len, table):
    lim = jnp.asarray(seq_len, jnp.int32).reshape(1) - 1
    out = _tc_copy(lim, table, table[:, None, :])
    return out[None]


# speedup vs baseline: 26.8333x; 1.2300x over previous
import functools

import jax
import jax.numpy as jnp
from jax import lax
from jax.experimental import pallas as pl
from jax.experimental.pallas import tpu as pltpu
from jax.experimental.pallas import tpu_sc as plsc

_B = 8192
_D = 2048
_NC = 2
_NS = 16
_NW = _NC * _NS
_ROWS_PER_W = _B // _NW
_CH = 16
_NCHUNK = _ROWS_PER_W // _CH
_NBUF = 3
_NSTAGE = 4
_STAGE_ROWS = _B // (_NC * _NSTAGE)
_SCH = 16
_SNBUF = 3
_SNCHUNK = _STAGE_ROWS // _SCH


def _pos_gather_body(lim_hbm, table_hbm, out_hbm, lim_v,
                     idx0, idx1, idx2, rows0, rows1, rows2, shared,
                     gsem0, gsem1, gsem2, ssem0, ssem1, ssem2):
    cid = lax.axis_index("c")
    sid = lax.axis_index("s")
    wid = sid * _NC + cid
    base = wid * _ROWS_PER_W

    pltpu.sync_copy(lim_hbm, lim_v)
    lim_vec = lim_v[...]
    limit = lim_vec[0]
    lanes = lax.iota(jnp.int32, _CH)

    idx_refs = (idx0, idx1, idx2)
    row_refs = (rows0, rows1, rows2)
    gsems = (gsem0, gsem1, gsem2)
    ssems = (ssem0, ssem1, ssem2)

    all_identity = limit >= _B - 1
    is_stager = sid % (_NS // _NSTAGE) == 0

    @pl.when(jnp.logical_and(all_identity, is_stager))
    def _fast():
        k = sid // (_NS // _NSTAGE)
        start = cid * (_B // _NC) + k * _STAGE_ROWS
        gathers = [None] * _SNBUF
        stores = [None] * _SNBUF

        def fire_load(g):
            slot = g % _SNBUF
            return pltpu.async_copy(
                table_hbm.at[pl.ds(start + g * _SCH, _SCH)],
                shared.at[k, slot], gsems[slot])

        for b in range(_SNBUF):
            gathers[b] = fire_load(b)
        for g in range(_SNCHUNK):
            slot = g % _SNBUF
            gathers[slot].wait()
            stores[slot] = pltpu.async_copy(
                shared.at[k, slot],
                out_hbm.at[pl.ds(start + g * _SCH, _SCH)],
                ssems[slot])
            nxt = g + _SNBUF
            if nxt < _SNCHUNK:
                stores[slot].wait()
                gathers[slot] = fire_load(nxt)
            elif g >= _SNCHUNK - _SNBUF:
                stores[slot].wait()

    @pl.when(jnp.logical_not(all_identity))
    def _general():
        def fire_gather(g):
            slot = g % _NBUF
            row0 = base + g * _CH
            idx_refs[slot][...] = jnp.minimum(row0 + lanes, lim_vec)
            return pltpu.async_copy(table_hbm.at[idx_refs[slot]],
                                    row_refs[slot], gsems[slot])

        gathers = [None] * _NBUF
        stores = [None] * _NBUF
        for b in range(_NBUF):
            gathers[b] = fire_gather(b)
        for g in range(_NCHUNK):
            slot = g % _NBUF
            gathers[slot].wait()
            stores[slot] = pltpu.async_copy(
                row_refs[slot], out_hbm.at[pl.ds(base + g * _CH, _CH)],
                ssems[slot])
            nxt = g + _NBUF
            if nxt < _NCHUNK:
                stores[slot].wait()
                gathers[slot] = fire_gather(nxt)
            elif g >= _NCHUNK - _NBUF:
                stores[slot].wait()


_pos_gather = functools.partial(
    pl.kernel,
    out_type=jax.ShapeDtypeStruct((_B, _D), jnp.float32),
    mesh=plsc.VectorSubcoreMesh(core_axis_name="c", subcore_axis_name="s"),
    scratch_types=[
        pltpu.VMEM((_CH,), jnp.int32),
        pltpu.VMEM((_CH,), jnp.int32),
        pltpu.VMEM((_CH,), jnp.int32),
        pltpu.VMEM((_CH,), jnp.int32),
        pltpu.VMEM((_CH, _D), jnp.float32),
        pltpu.VMEM((_CH, _D), jnp.float32),
        pltpu.VMEM((_CH, _D), jnp.float32),
        pltpu.VMEM_SHARED((_NSTAGE, _SNBUF, _SCH, _D), jnp.float32),
        pltpu.SemaphoreType.DMA,
        pltpu.SemaphoreType.DMA,
        pltpu.SemaphoreType.DMA,
        pltpu.SemaphoreType.DMA,
        pltpu.SemaphoreType.DMA,
        pltpu.SemaphoreType.DMA,
    ],
)(_pos_gather_body)


def kernel(seq_len, table):
    lim = jnp.full((_CH,), jnp.asarray(seq_len, jnp.int32) - 1, jnp.int32)
    out = _pos_gather(lim, table)
    return out[None]


# speedup vs baseline: 29.2394x; 1.0897x over previous
import functools

import jax
import jax.numpy as jnp
from jax import lax
from jax.experimental import pallas as pl
from jax.experimental.pallas import tpu as pltpu
from jax.experimental.pallas import tpu_sc as plsc

_B = 8192
_D = 2048
_NC = 2
_NS = 16
_NW = _NC * _NS
_ROWS_PER_W = 256
_CH = 16
_NCHUNK = _ROWS_PER_W // _CH
_NBUF = 3
_NSTAGE = 4
_STAGE_BASE = (_NW - _NSTAGE * _NC) * _ROWS_PER_W


def _pos_gather_body(lim_hbm, table_hbm, out_hbm, lim_v,
                     idx0, idx1, idx2, rows0, rows1, rows2, shared,
                     gsem0, gsem1, gsem2, ssem0, ssem1, ssem2):
    cid = lax.axis_index("c")
    sid = lax.axis_index("s")
    wid = sid * _NC + cid
    base = wid * _ROWS_PER_W

    pltpu.sync_copy(lim_hbm, lim_v)
    lim_vec = lim_v[...]
    limit = lim_vec[0]
    lanes = lax.iota(jnp.int32, _CH)

    idx_refs = (idx0, idx1, idx2)
    row_refs = (rows0, rows1, rows2)
    gsems = (gsem0, gsem1, gsem2)
    ssems = (ssem0, ssem1, ssem2)

    def run_ring(ring_base):
        def fire_gather(g):
            slot = g % _NBUF
            row0 = ring_base + g * _CH
            idx_refs[slot][...] = jnp.minimum(row0 + lanes, lim_vec)
            return pltpu.async_copy(table_hbm.at[idx_refs[slot]],
                                    row_refs[slot], gsems[slot])

        gathers = [None] * _NBUF
        stores = [None] * _NBUF
        for b in range(_NBUF):
            gathers[b] = fire_gather(b)
        for g in range(_NCHUNK):
            slot = g % _NBUF
            gathers[slot].wait()
            stores[slot] = pltpu.async_copy(
                row_refs[slot],
                out_hbm.at[pl.ds(ring_base + g * _CH, _CH)], ssems[slot])
            nxt = g + _NBUF
            if nxt < _NCHUNK:
                stores[slot].wait()
                gathers[slot] = fire_gather(nxt)
            elif g >= _NCHUNK - _NBUF:
                stores[slot].wait()

    all_identity = limit >= _B - 1
    is_stager = sid % (_NS // _NSTAGE) == 0

    @pl.when(jnp.logical_and(all_identity, is_stager))
    def _fast_stage():
        k = sid // (_NS // _NSTAGE)
        stager_id = k * _NC + cid
        start = _STAGE_BASE + stager_id * _ROWS_PER_W
        gathers = [None] * _NBUF
        stores = [None] * _NBUF

        def fire_load(g):
            slot = g % _NBUF
            return pltpu.async_copy(
                table_hbm.at[pl.ds(start + g * _CH, _CH)],
                shared.at[k, slot], gsems[slot])

        for b in range(_NBUF):
            gathers[b] = fire_load(b)
        for g in range(_NCHUNK):
            slot = g % _NBUF
            gathers[slot].wait()
            stores[slot] = pltpu.async_copy(
                shared.at[k, slot],
                out_hbm.at[pl.ds(start + g * _CH, _CH)], ssems[slot])
            nxt = g + _NBUF
            if nxt < _NCHUNK:
                stores[slot].wait()
                gathers[slot] = fire_load(nxt)
            elif g >= _NCHUNK - _NBUF:
                stores[slot].wait()

    @pl.when(jnp.logical_and(all_identity, jnp.logical_not(is_stager)))
    def _fast_stream():
        nsid = sid - sid // (_NS // _NSTAGE) - 1
        run_ring((nsid * _NC + cid) * _ROWS_PER_W)

    @pl.when(jnp.logical_not(all_identity))
    def _general():
        run_ring(base)


_pos_gather = functools.partial(
    pl.kernel,
    out_type=jax.ShapeDtypeStruct((_B, _D), jnp.float32),
    mesh=plsc.VectorSubcoreMesh(core_axis_name="c", subcore_axis_name="s"),
    scratch_types=[
        pltpu.VMEM((_CH,), jnp.int32),
        pltpu.VMEM((_CH,), jnp.int32),
        pltpu.VMEM((_CH,), jnp.int32),
        pltpu.VMEM((_CH,), jnp.int32),
        pltpu.VMEM((_CH, _D), jnp.float32),
        pltpu.VMEM((_CH, _D), jnp.float32),
        pltpu.VMEM((_CH, _D), jnp.float32),
        pltpu.VMEM_SHARED((_NSTAGE, _NBUF, _CH, _D), jnp.float32),
        pltpu.SemaphoreType.DMA,
        pltpu.SemaphoreType.DMA,
        pltpu.SemaphoreType.DMA,
        pltpu.SemaphoreType.DMA,
        pltpu.SemaphoreType.DMA,
        pltpu.SemaphoreType.DMA,
    ],
)(_pos_gather_body)


def kernel(seq_len, table):
    lim = jnp.full((_CH,), jnp.asarray(seq_len, jnp.int32) - 1, jnp.int32)
    out = _pos_gather(lim, table)
    return out[None]
